# baseline (speedup 1.0000x reference)
import functools

import jax
import jax.numpy as jnp
from jax import lax
from jax.experimental import pallas as pl
from jax.experimental.pallas import tpu as pltpu
from jax.experimental.pallas import tpu_sc as plsc

T, D, E, FF, K = 4096, 1024, 8, 4096, 2
EPAD = 128
BT = 512
NB = (T * K) // BT
W_PAD = 32
NEG = -1e30

NC, NS, L = 2, 16, 16
NW = NC * NS
TPW = T // NW
PPW = (T * K) // NW

_MESH = plsc.VectorSubcoreMesh(core_axis_name="c", subcore_axis_name="s")
_SC_PARAMS = pltpu.CompilerParams(needs_layout_passes=False)


def _wid():
    return lax.axis_index("s") * NC + lax.axis_index("c")


def _gate_body(x_ref, wg_ref, bg_ref, i1_ref, i2_ref, s1_ref, s2_ref):
    x = x_ref[...]
    logits = jnp.dot(x, wg_ref[...], preferred_element_type=jnp.float32)
    logits = logits + bg_ref[...]
    lane = lax.broadcasted_iota(jnp.int32, logits.shape, 1)
    v1 = jnp.max(logits, axis=-1, keepdims=True)
    i1 = jnp.min(jnp.where(logits == v1, lane, EPAD), axis=-1, keepdims=True)
    l2 = jnp.where(lane == i1, NEG, logits)
    v2 = jnp.max(l2, axis=-1, keepdims=True)
    i2 = jnp.min(jnp.where(l2 == v2, lane, EPAD), axis=-1, keepdims=True)
    e2 = jnp.exp(v2 - v1)
    s1_ref[...] = 1.0 / (1.0 + e2)
    s2_ref[...] = e2 / (1.0 + e2)
    i1_ref[...] = i1
    i2_ref[...] = i2


def _gate(x, wg, bg):
    btg = 1024
    wgp = jnp.zeros((D, EPAD), jnp.float32).at[:, :E].set(wg)
    bgp = jnp.full((EPAD,), NEG, jnp.float32).at[:E].set(bg).reshape(1, EPAD)
    return pl.pallas_call(
        _gate_body,
        grid=(T // btg,),
        in_specs=[
            pl.BlockSpec((btg, D), lambda i: (i, 0)),
            pl.BlockSpec((D, EPAD), lambda i: (0, 0)),
            pl.BlockSpec((1, EPAD), lambda i: (0, 0)),
        ],
        out_specs=[pl.BlockSpec((btg, 1), lambda i: (i, 0))] * 4,
        out_shape=[
            jax.ShapeDtypeStruct((T, 1), jnp.int32),
            jax.ShapeDtypeStruct((T, 1), jnp.int32),
            jax.ShapeDtypeStruct((T, 1), jnp.float32),
            jax.ShapeDtypeStruct((T, 1), jnp.float32),
        ],
    )(x, wgp, bgp)


@functools.partial(
    pl.kernel, mesh=_MESH, compiler_params=_SC_PARAMS,
    out_type=jax.ShapeDtypeStruct((NW, L), jnp.int32),
    scratch_types=[
        pltpu.VMEM((TPW,), jnp.int32),
        pltpu.VMEM((TPW,), jnp.int32),
        pltpu.VMEM((L,), jnp.int32),
    ],
)
def _sc_hist(idx0_hbm, idx1_hbm, hist_hbm, idx0_v, idx1_v, hist_v):
    wid = _wid()
    base = wid * TPW
    pltpu.sync_copy(idx0_hbm.at[pl.ds(base, TPW)], idx0_v)
    pltpu.sync_copy(idx1_hbm.at[pl.ds(base, TPW)], idx1_v)
    lane = lax.iota(jnp.int32, L)
    one = jnp.ones((L,), jnp.int32)
    zero = jnp.zeros((L,), jnp.int32)
    hist = zero
    for ref in (idx0_v, idx1_v):
        for j in range(TPW // L):
            v = ref[pl.ds(j * L, L)]
            for e in range(E):
                cnt = jnp.sum(jnp.where(v == e, one, zero))
                hist = jnp.where(lane == e, hist + cnt, hist)
    hist_v[...] = hist
    pltpu.sync_copy(hist_v, hist_hbm.at[wid])


@functools.partial(
    pl.kernel, mesh=_MESH, compiler_params=_SC_PARAMS,
    out_type=[
        jax.ShapeDtypeStruct((T * K, D), jnp.float32),
        jax.ShapeDtypeStruct((T,), jnp.int32),
        jax.ShapeDtypeStruct((T,), jnp.int32),
        jax.ShapeDtypeStruct((L,), jnp.int32),
        jax.ShapeDtypeStruct((W_PAD,), jnp.int32),
        jax.ShapeDtypeStruct((W_PAD,), jnp.int32),
    ],
    scratch_types=[
        pltpu.VMEM((TPW,), jnp.int32),
        pltpu.VMEM((TPW,), jnp.int32),
        pltpu.VMEM((NW, L), jnp.int32),
        pltpu.VMEM((L,), jnp.int32),
        pltpu.VMEM((L,), jnp.int32),
        pltpu.VMEM((L,), jnp.int32),
        pltpu.VMEM((NB,), jnp.int32),
        pltpu.VMEM((NB,), jnp.int32),
        pltpu.VMEM((4, TPW // 4), jnp.int32),
        pltpu.VMEM((4, TPW // 4), jnp.int32),
        pltpu.VMEM((TPW // 4, D), jnp.float32),
        pltpu.VMEM((L,), jnp.int32),
        pltpu.VMEM((W_PAD,), jnp.int32),
        pltpu.VMEM((W_PAD,), jnp.int32),
        pltpu.SemaphoreType.DMA,
    ],
)
def _sc_dispatch(x_hbm, idx0_hbm, idx1_hbm, hist_hbm,
                 xs_hbm, inv0_hbm, inv1_hbm, meta_hbm, wb_hbm, we_hbm,
                 idx0_v, idx1_v, hist_v, myB_v, offs_v, tmp_v, fe_v, se_v,
                 dst0_v, dst1_v, xrow_v, meta_v, wb_v, we_v, sem):
    wid = _wid()
    base = wid * TPW
    pltpu.sync_copy(hist_hbm, hist_v)
    pltpu.sync_copy(idx0_hbm.at[pl.ds(base, TPW)], idx0_v)
    pltpu.sync_copy(idx1_hbm.at[pl.ds(base, TPW)], idx1_v)
    lane = lax.iota(jnp.int32, L)
    zero = jnp.zeros((L,), jnp.int32)
    one = jnp.ones((L,), jnp.int32)

    def _acc(w, acc):
        return acc + hist_v[w]

    prefix = lax.fori_loop(0, wid, _acc, zero)
    total = lax.fori_loop(0, NW, _acc, zero)
    o_incl = jnp.cumsum(total)
    o_excl = o_incl - total
    offs = jnp.where(lane < E, o_excl, o_incl)
    offs_v[...] = offs
    myB_v[...] = o_excl + prefix

    for idxref, dstref in ((idx0_v, dst0_v), (idx1_v, dst1_v)):
        for j in range(TPW // L):
            v = idxref[pl.ds(j * L, L)]
            basevec = plsc.load_gather(myB_v, [v])
            rank = zero
            upd = myB_v[...]
            for e in range(E):
                m = v == e
                c = jnp.cumsum(jnp.where(m, one, zero))
                rank = jnp.where(m, c - 1, rank)
                cnt = jnp.sum(jnp.where(m, one, zero))
                upd = jnp.where(lane == e, upd + cnt, upd)
            myB_v[...] = upd
            dstref[j // 2, pl.ds((j % 2) * L, L)] = basevec + rank

    cs = TPW // 4
    for c in range(4):
        pltpu.sync_copy(x_hbm.at[pl.ds(base + c * cs, cs)], xrow_v)
        cp0 = pltpu.async_copy(xrow_v, xs_hbm.at[dst0_v.at[c]], sem)
        cp1 = pltpu.async_copy(xrow_v, xs_hbm.at[dst1_v.at[c]], sem)
        pltpu.sync_copy(dst0_v.at[c], inv0_hbm.at[pl.ds(base + c * cs, cs)])
        pltpu.sync_copy(dst1_v.at[c], inv1_hbm.at[pl.ds(base + c * cs, cs)])
        cp0.wait()
        cp1.wait()

    @pl.when(wid == 0)
    def _worklist():
        nb_half = []
        for half in range(NB // L):
            bvec = lax.iota(jnp.int32, L) + L * half
            r_lo = bvec * BT
            r_hi = r_lo + (BT - 1)
            fe = zero
            le = zero
            for e in range(1, E):
                sp = plsc.load_gather(offs_v, [jnp.full((L,), e, jnp.int32)])
                fe = fe + jnp.where(sp <= r_lo, one, zero)
                le = le + jnp.where(sp <= r_hi, one, zero)
            fe_v[pl.ds(half * L, L)] = fe
            nb_half.append(le - fe + 1)
        carry = None
        for half in range(NB // L):
            c = jnp.cumsum(nb_half[half])
            if carry is not None:
                c = c + carry
            se_v[pl.ds(half * L, L)] = c - nb_half[half]
            tmp_v[...] = c
            carry = plsc.load_gather(tmp_v, [jnp.full((L,), L - 1, jnp.int32)])
        meta_v[...] = jnp.where(lane == L - 1, carry, offs)
        for h in range(W_PAD // L):
            wv = lax.iota(jnp.int32, L) + L * h
            bo = zero
            for b in range(1, NB):
                spb = plsc.load_gather(se_v, [jnp.full((L,), b, jnp.int32)])
                bo = bo + jnp.where(spb <= wv, one, zero)
            fe_g = plsc.load_gather(fe_v, [bo])
            se_g = plsc.load_gather(se_v, [bo])
            eo = jnp.clip(fe_g + (wv - se_g), 0, E - 1)
            wb_v[pl.ds(h * L, L)] = bo
            we_v[pl.ds(h * L, L)] = eo
        pltpu.sync_copy(meta_v, meta_hbm)
        pltpu.sync_copy(wb_v, wb_hbm)
        pltpu.sync_copy(we_v, we_hbm)


def _ffn_body(wb, we, meta, xs_ref, w1_ref, b1_ref, w2_ref, b2_ref, out_ref):
    w = pl.program_id(0)
    b = wb[w]
    first = jnp.logical_or(w == 0, wb[jnp.maximum(w - 1, 0)] != b)

    @pl.when(first)
    def _init():
        out_ref[...] = jnp.zeros_like(out_ref)

    @pl.when(w < meta[L - 1])
    def _compute():
        e = we[w]
        lo = meta[e]
        hi = meta[e + 1]
        rows = b * BT + lax.broadcasted_iota(jnp.int32, (BT, 1), 0)
        m = jnp.logical_and(rows >= lo, rows < hi).astype(jnp.float32)
        x = xs_ref[...].astype(jnp.bfloat16)
        h = jnp.dot(x, w1_ref[0], preferred_element_type=jnp.float32)
        h = jnp.maximum(h + b1_ref[0], 0.0).astype(jnp.bfloat16)
        y = jnp.dot(h, w2_ref[0], preferred_element_type=jnp.float32)
        out_ref[...] += m * (y + b2_ref[0])


def _ffn(xs, w1, b1, w2, b2, wb, we, meta):
    grid_spec = pltpu.PrefetchScalarGridSpec(
        num_scalar_prefetch=3,
        grid=(W_PAD,),
        in_specs=[
            pl.BlockSpec((BT, D), lambda w, wb, we, meta: (wb[w], 0)),
            pl.BlockSpec((1, D, FF), lambda w, wb, we, meta: (we[w], 0, 0)),
            pl.BlockSpec((1, 1, FF), lambda w, wb, we, meta: (we[w], 0, 0)),
            pl.BlockSpec((1, FF, D), lambda w, wb, we, meta: (we[w], 0, 0)),
            pl.BlockSpec((1, 1, D), lambda w, wb, we, meta: (we[w], 0, 0)),
        ],
        out_specs=pl.BlockSpec((BT, D), lambda w, wb, we, meta: (wb[w], 0)),
    )
    return pl.pallas_call(
        _ffn_body,
        grid_spec=grid_spec,
        out_shape=jax.ShapeDtypeStruct((T * K, D), jnp.float32),
        compiler_params=pltpu.CompilerParams(
            dimension_semantics=("arbitrary",),
        ),
    )(wb, we, meta, xs, w1, b1, w2, b2)


@functools.partial(
    pl.kernel, mesh=_MESH, compiler_params=_SC_PARAMS,
    out_type=jax.ShapeDtypeStruct((T, D), jnp.float32),
    scratch_types=[
        pltpu.VMEM((TPW // 4,), jnp.int32),
        pltpu.VMEM((TPW // 4,), jnp.int32),
        pltpu.VMEM((TPW // 4 + 8,), jnp.float32),
        pltpu.VMEM((TPW // 4 + 8,), jnp.float32),
        pltpu.VMEM((TPW // 4, D), jnp.float32),
        pltpu.VMEM((TPW // 4, D), jnp.float32),
        pltpu.SemaphoreType.DMA,
    ],
)
def _sc_combine(yw_hbm, inv0_hbm, inv1_hbm, ws0_hbm, ws1_hbm, out_hbm,
                i0_v, i1_v, w0_v, w1_v, r0_v, r1_v, sem):
    wid = _wid()
    cs = TPW // 4

    def _chunk(c, _):
        tb = wid * TPW + c * cs
        pltpu.sync_copy(inv0_hbm.at[pl.ds(tb, cs)], i0_v)
        pltpu.sync_copy(inv1_hbm.at[pl.ds(tb, cs)], i1_v)
        pltpu.sync_copy(ws0_hbm.at[pl.ds(tb, cs)], w0_v.at[pl.ds(8, cs)])
        pltpu.sync_copy(ws1_hbm.at[pl.ds(tb, cs)], w1_v.at[pl.ds(8, cs)])
        cp0 = pltpu.async_copy(yw_hbm.at[i0_v], r0_v, sem)
        cp1 = pltpu.async_copy(yw_hbm.at[i1_v], r1_v, sem)
        cp0.wait()
        cp1.wait()
        for r in range(cs):
            s0 = plsc.load_gather(w0_v, [jnp.full((L,), r + 8, jnp.int32)])
            s1 = plsc.load_gather(w1_v, [jnp.full((L,), r + 8, jnp.int32)])

            @plsc.parallel_loop(0, D // L, unroll=8)
            def _mix(k, r=r, s0=s0, s1=s1):
                a = r0_v[r, pl.ds(k * L, L)]
                bvec = r1_v[r, pl.ds(k * L, L)]
                r0_v[r, pl.ds(k * L, L)] = s0 * a + s1 * bvec

        pltpu.sync_copy(r0_v, out_hbm.at[pl.ds(tb, cs)])
        return 0

    lax.fori_loop(0, 4, _chunk, 0)


def kernel(moe_inp, Wg, bg, W1, b1, W2, b2):
    i1, i2, s1, s2 = _gate(moe_inp, Wg, bg)
    idx0 = i1.reshape(T)
    idx1 = i2.reshape(T)
    ws0 = s1.reshape(T)
    ws1 = s2.reshape(T)
    hist = _sc_hist(idx0, idx1)
    xs, inv0, inv1, meta, wb, we = _sc_dispatch(moe_inp, idx0, idx1, hist)
    yw = _ffn(xs, W1.astype(jnp.bfloat16), b1.reshape(E, 1, FF),
              W2.astype(jnp.bfloat16), b2.reshape(E, 1, D), wb, we, meta)
    return _sc_combine(yw, inv0, inv1, ws0, ws1)


# speedup vs baseline: 1.0332x; 1.0332x over previous
import functools

import jax
import jax.numpy as jnp
from jax import lax
from jax.experimental import pallas as pl
from jax.experimental.pallas import tpu as pltpu
from jax.experimental.pallas import tpu_sc as plsc

T, D, E, FF, K = 4096, 1024, 8, 4096, 2
EPAD = 128
BT = 256
NB = (T * K) // BT
W_PAD = 48
NEG = -1e30

NC, NS, L = 2, 16, 16
NW = NC * NS
TPW = T // NW
PPW = (T * K) // NW

_MESH = plsc.VectorSubcoreMesh(core_axis_name="c", subcore_axis_name="s")
_SC_PARAMS = pltpu.CompilerParams(needs_layout_passes=False)


def _wid():
    return lax.axis_index("s") * NC + lax.axis_index("c")


def _gate_body(x_ref, wg_ref, bg_ref, i1_ref, i2_ref, s1_ref, s2_ref):
    x = x_ref[...]
    logits = jnp.dot(x, wg_ref[...], preferred_element_type=jnp.float32)
    logits = logits + bg_ref[...]
    lane = lax.broadcasted_iota(jnp.int32, logits.shape, 1)
    v1 = jnp.max(logits, axis=-1, keepdims=True)
    i1 = jnp.min(jnp.where(logits == v1, lane, EPAD), axis=-1, keepdims=True)
    l2 = jnp.where(lane == i1, NEG, logits)
    v2 = jnp.max(l2, axis=-1, keepdims=True)
    i2 = jnp.min(jnp.where(l2 == v2, lane, EPAD), axis=-1, keepdims=True)
    e2 = jnp.exp(v2 - v1)
    s1_ref[...] = 1.0 / (1.0 + e2)
    s2_ref[...] = e2 / (1.0 + e2)
    i1_ref[...] = i1
    i2_ref[...] = i2


def _gate(x, wg, bg):
    btg = 1024
    wgp = jnp.zeros((D, EPAD), jnp.float32).at[:, :E].set(wg)
    bgp = jnp.full((EPAD,), NEG, jnp.float32).at[:E].set(bg).reshape(1, EPAD)
    return pl.pallas_call(
        _gate_body,
        grid=(T // btg,),
        in_specs=[
            pl.BlockSpec((btg, D), lambda i: (i, 0)),
            pl.BlockSpec((D, EPAD), lambda i: (0, 0)),
            pl.BlockSpec((1, EPAD), lambda i: (0, 0)),
        ],
        out_specs=[pl.BlockSpec((btg, 1), lambda i: (i, 0))] * 4,
        out_shape=[
            jax.ShapeDtypeStruct((T, 1), jnp.int32),
            jax.ShapeDtypeStruct((T, 1), jnp.int32),
            jax.ShapeDtypeStruct((T, 1), jnp.float32),
            jax.ShapeDtypeStruct((T, 1), jnp.float32),
        ],
    )(x, wgp, bgp)


@functools.partial(
    pl.kernel, mesh=_MESH, compiler_params=_SC_PARAMS,
    out_type=jax.ShapeDtypeStruct((NW, L), jnp.int32),
    scratch_types=[
        pltpu.VMEM((TPW,), jnp.int32),
        pltpu.VMEM((TPW,), jnp.int32),
        pltpu.VMEM((L,), jnp.int32),
    ],
)
def _sc_hist(idx0_hbm, idx1_hbm, hist_hbm, idx0_v, idx1_v, hist_v):
    wid = _wid()
    base = wid * TPW
    pltpu.sync_copy(idx0_hbm.at[pl.ds(base, TPW)], idx0_v)
    pltpu.sync_copy(idx1_hbm.at[pl.ds(base, TPW)], idx1_v)
    lane = lax.iota(jnp.int32, L)
    one = jnp.ones((L,), jnp.int32)
    zero = jnp.zeros((L,), jnp.int32)
    hist = zero
    for ref in (idx0_v, idx1_v):
        for j in range(TPW // L):
            v = ref[pl.ds(j * L, L)]
            for e in range(E):
                cnt = jnp.sum(jnp.where(v == e, one, zero))
                hist = jnp.where(lane == e, hist + cnt, hist)
    hist_v[...] = hist
    pltpu.sync_copy(hist_v, hist_hbm.at[wid])


@functools.partial(
    pl.kernel, mesh=_MESH, compiler_params=_SC_PARAMS,
    out_type=[
        jax.ShapeDtypeStruct((T * K, D), jnp.float32),
        jax.ShapeDtypeStruct((T,), jnp.int32),
        jax.ShapeDtypeStruct((T,), jnp.int32),
        jax.ShapeDtypeStruct((L,), jnp.int32),
        jax.ShapeDtypeStruct((W_PAD,), jnp.int32),
        jax.ShapeDtypeStruct((W_PAD,), jnp.int32),
    ],
    scratch_types=[
        pltpu.VMEM((TPW,), jnp.int32),
        pltpu.VMEM((TPW,), jnp.int32),
        pltpu.VMEM((NW, L), jnp.int32),
        pltpu.VMEM((L,), jnp.int32),
        pltpu.VMEM((L,), jnp.int32),
        pltpu.VMEM((L,), jnp.int32),
        pltpu.VMEM((NB,), jnp.int32),
        pltpu.VMEM((NB,), jnp.int32),
        pltpu.VMEM((4, TPW // 4), jnp.int32),
        pltpu.VMEM((4, TPW // 4), jnp.int32),
        pltpu.VMEM((2, TPW // 4, D), jnp.float32),
        pltpu.VMEM((L,), jnp.int32),
        pltpu.VMEM((W_PAD,), jnp.int32),
        pltpu.VMEM((W_PAD,), jnp.int32),
        pltpu.SemaphoreType.DMA,
        pltpu.SemaphoreType.DMA,
    ],
)
def _sc_dispatch(x_hbm, idx0_hbm, idx1_hbm, hist_hbm,
                 xs_hbm, inv0_hbm, inv1_hbm, meta_hbm, wb_hbm, we_hbm,
                 idx0_v, idx1_v, hist_v, myB_v, offs_v, tmp_v, fe_v, se_v,
                 dst0_v, dst1_v, xrow_v, meta_v, wb_v, we_v, sem, rsem):
    wid = _wid()
    base = wid * TPW
    pltpu.sync_copy(hist_hbm, hist_v)
    pltpu.sync_copy(idx0_hbm.at[pl.ds(base, TPW)], idx0_v)
    pltpu.sync_copy(idx1_hbm.at[pl.ds(base, TPW)], idx1_v)
    lane = lax.iota(jnp.int32, L)
    zero = jnp.zeros((L,), jnp.int32)
    one = jnp.ones((L,), jnp.int32)

    def _acc(w, acc):
        return acc + hist_v[w]

    prefix = lax.fori_loop(0, wid, _acc, zero)
    total = lax.fori_loop(0, NW, _acc, zero)
    o_incl = jnp.cumsum(total)
    o_excl = o_incl - total
    offs = jnp.where(lane < E, o_excl, o_incl)
    offs_v[...] = offs
    myB_v[...] = o_excl + prefix

    for idxref, dstref in ((idx0_v, dst0_v), (idx1_v, dst1_v)):
        for j in range(TPW // L):
            v = idxref[pl.ds(j * L, L)]
            basevec = plsc.load_gather(myB_v, [v])
            rank = zero
            upd = myB_v[...]
            for e in range(E):
                m = v == e
                c = jnp.cumsum(jnp.where(m, one, zero))
                rank = jnp.where(m, c - 1, rank)
                cnt = jnp.sum(jnp.where(m, one, zero))
                upd = jnp.where(lane == e, upd + cnt, upd)
            myB_v[...] = upd
            dstref[j // 2, pl.ds((j % 2) * L, L)] = basevec + rank

    cs = TPW // 4
    rd = pltpu.async_copy(x_hbm.at[pl.ds(base, cs)], xrow_v.at[0], rsem)
    for c in range(4):
        rd.wait()
        if c < 3:
            rd = pltpu.async_copy(
                x_hbm.at[pl.ds(base + (c + 1) * cs, cs)], xrow_v.at[(c + 1) % 2], rsem)
        cp0 = pltpu.async_copy(xrow_v.at[c % 2], xs_hbm.at[dst0_v.at[c]], sem)
        cp1 = pltpu.async_copy(xrow_v.at[c % 2], xs_hbm.at[dst1_v.at[c]], sem)
        pltpu.sync_copy(dst0_v.at[c], inv0_hbm.at[pl.ds(base + c * cs, cs)])
        pltpu.sync_copy(dst1_v.at[c], inv1_hbm.at[pl.ds(base + c * cs, cs)])
        cp0.wait()
        cp1.wait()

    @pl.when(wid == 0)
    def _worklist():
        nb_half = []
        for half in range(NB // L):
            bvec = lax.iota(jnp.int32, L) + L * half
            r_lo = bvec * BT
            r_hi = r_lo + (BT - 1)
            fe = zero
            le = zero
            for e in range(1, E):
                sp = plsc.load_gather(offs_v, [jnp.full((L,), e, jnp.int32)])
                fe = fe + jnp.where(sp <= r_lo, one, zero)
                le = le + jnp.where(sp <= r_hi, one, zero)
            fe_v[pl.ds(half * L, L)] = fe
            nb_half.append(le - fe + 1)
        carry = None
        for half in range(NB // L):
            c = jnp.cumsum(nb_half[half])
            if carry is not None:
                c = c + carry
            se_v[pl.ds(half * L, L)] = c - nb_half[half]
            tmp_v[...] = c
            carry = plsc.load_gather(tmp_v, [jnp.full((L,), L - 1, jnp.int32)])
        meta_v[...] = jnp.where(lane == L - 1, carry, offs)
        for h in range(W_PAD // L):
            wv = lax.iota(jnp.int32, L) + L * h
            bo = zero
            for b in range(1, NB):
                spb = plsc.load_gather(se_v, [jnp.full((L,), b, jnp.int32)])
                bo = bo + jnp.where(spb <= wv, one, zero)
            fe_g = plsc.load_gather(fe_v, [bo])
            se_g = plsc.load_gather(se_v, [bo])
            eo = jnp.clip(fe_g + (wv - se_g), 0, E - 1)
            wb_v[pl.ds(h * L, L)] = bo
            we_v[pl.ds(h * L, L)] = eo
        pltpu.sync_copy(meta_v, meta_hbm)
        pltpu.sync_copy(wb_v, wb_hbm)
        pltpu.sync_copy(we_v, we_hbm)


def _ffn_body(wb, we, meta, xs_ref, w1_ref, b1_ref, w2_ref, b2_ref, out_ref):
    w = pl.program_id(0)
    b = wb[w]
    first = jnp.logical_or(w == 0, wb[jnp.maximum(w - 1, 0)] != b)

    @pl.when(first)
    def _init():
        out_ref[...] = jnp.zeros_like(out_ref)

    @pl.when(w < meta[L - 1])
    def _compute():
        e = we[w]
        lo = meta[e]
        hi = meta[e + 1]
        rows = b * BT + lax.broadcasted_iota(jnp.int32, (BT, 1), 0)
        m = jnp.logical_and(rows >= lo, rows < hi).astype(jnp.float32)
        x = xs_ref[...].astype(jnp.bfloat16)
        h = jnp.dot(x, w1_ref[0], preferred_element_type=jnp.float32)
        h = jnp.maximum(h + b1_ref[0], 0.0).astype(jnp.bfloat16)
        y = jnp.dot(h, w2_ref[0], preferred_element_type=jnp.float32)
        out_ref[...] += m * (y + b2_ref[0])


def _ffn(xs, w1, b1, w2, b2, wb, we, meta):
    grid_spec = pltpu.PrefetchScalarGridSpec(
        num_scalar_prefetch=3,
        grid=(W_PAD,),
        in_specs=[
            pl.BlockSpec((BT, D), lambda w, wb, we, meta: (wb[w], 0)),
            pl.BlockSpec((1, D, FF), lambda w, wb, we, meta: (we[w], 0, 0)),
            pl.BlockSpec((1, 1, FF), lambda w, wb, we, meta: (we[w], 0, 0)),
            pl.BlockSpec((1, FF, D), lambda w, wb, we, meta: (we[w], 0, 0)),
            pl.BlockSpec((1, 1, D), lambda w, wb, we, meta: (we[w], 0, 0)),
        ],
        out_specs=pl.BlockSpec((BT, D), lambda w, wb, we, meta: (wb[w], 0)),
    )
    return pl.pallas_call(
        _ffn_body,
        grid_spec=grid_spec,
        out_shape=jax.ShapeDtypeStruct((T * K, D), jnp.float32),
        compiler_params=pltpu.CompilerParams(
            dimension_semantics=("arbitrary",),
        ),
    )(wb, we, meta, xs, w1, b1, w2, b2)


@functools.partial(
    pl.kernel, mesh=_MESH, compiler_params=_SC_PARAMS,
    out_type=jax.ShapeDtypeStruct((T, D), jnp.float32),
    scratch_types=[
        pltpu.VMEM((TPW // 4,), jnp.int32),
        pltpu.VMEM((TPW // 4,), jnp.int32),
        pltpu.VMEM((TPW // 4 + 8,), jnp.float32),
        pltpu.VMEM((TPW // 4 + 8,), jnp.float32),
        pltpu.VMEM((TPW // 4, D), jnp.float32),
        pltpu.VMEM((TPW // 4, D), jnp.float32),
        pltpu.SemaphoreType.DMA,
    ],
)
def _sc_combine(yw_hbm, inv0_hbm, inv1_hbm, ws0_hbm, ws1_hbm, out_hbm,
                i0_v, i1_v, w0_v, w1_v, r0_v, r1_v, sem):
    wid = _wid()
    cs = TPW // 4

    def _chunk(c, _):
        tb = wid * TPW + c * cs
        pltpu.sync_copy(inv0_hbm.at[pl.ds(tb, cs)], i0_v)
        pltpu.sync_copy(inv1_hbm.at[pl.ds(tb, cs)], i1_v)
        pltpu.sync_copy(ws0_hbm.at[pl.ds(tb, cs)], w0_v.at[pl.ds(8, cs)])
        pltpu.sync_copy(ws1_hbm.at[pl.ds(tb, cs)], w1_v.at[pl.ds(8, cs)])
        cp0 = pltpu.async_copy(yw_hbm.at[i0_v], r0_v, sem)
        cp1 = pltpu.async_copy(yw_hbm.at[i1_v], r1_v, sem)
        cp0.wait()
        cp1.wait()
        for r in range(cs):
            s0 = plsc.load_gather(w0_v, [jnp.full((L,), r + 8, jnp.int32)])
            s1 = plsc.load_gather(w1_v, [jnp.full((L,), r + 8, jnp.int32)])

            @plsc.parallel_loop(0, D // L, unroll=8)
            def _mix(k, r=r, s0=s0, s1=s1):
                a = r0_v[r, pl.ds(k * L, L)]
                bvec = r1_v[r, pl.ds(k * L, L)]
                r0_v[r, pl.ds(k * L, L)] = s0 * a + s1 * bvec

        pltpu.sync_copy(r0_v, out_hbm.at[pl.ds(tb, cs)])
        return 0

    lax.fori_loop(0, 4, _chunk, 0)


def kernel(moe_inp, Wg, bg, W1, b1, W2, b2):
    i1, i2, s1, s2 = _gate(moe_inp, Wg, bg)
    idx0 = i1.reshape(T)
    idx1 = i2.reshape(T)
    ws0 = s1.reshape(T)
    ws1 = s2.reshape(T)
    hist = _sc_hist(idx0, idx1)
    xs, inv0, inv1, meta, wb, we = _sc_dispatch(moe_inp, idx0, idx1, hist)
    yw = _ffn(xs, W1.astype(jnp.bfloat16), b1.reshape(E, 1, FF),
              W2.astype(jnp.bfloat16), b2.reshape(E, 1, D), wb, we, meta)
    return _sc_combine(yw, inv0, inv1, ws0, ws1)
